# baseline (speedup 1.0000x reference)
import functools

import jax
import jax.numpy as jnp
import numpy as np
from jax.experimental import pallas as pl
from jax.experimental.pallas import tpu as pltpu

EPS = 1e-6
K_NBRS = 20
HID = 128
NEG_BIG = -3.0e38


def _knn_convpos_kernel(p_ref, pt_ref, wd9t_ref, wp9t_ref, qt_ref,
                        o0_ref, o1_ref, o2_ref, *, n, k, bb):
    for bi in range(bb):
        p2d = p_ref[bi]
        pT = pt_ref[bi]

        inner = jnp.dot(p2d.astype(jnp.bfloat16), pT.astype(jnp.bfloat16),
                        preferred_element_type=jnp.float32)
        xx = pT * pT
        xxs = (xx[0:1, :] + xx[1:2, :]) + xx[2:3, :]
        xxc = jnp.sum(p2d * p2d, axis=1, keepdims=True)
        s = 2.0 * inner - xxc - xxs

        ctr0 = pT[0:1, :]
        ctr1 = pT[1:2, :]
        ctr2 = pT[2:3, :]

        acc = jnp.zeros((192, n), jnp.float32)
        xpool = jnp.zeros((9, n), jnp.float32)
        for _ in range(k):
            mx = jnp.max(s, axis=0, keepdims=True)
            eq = s == mx
            oh = eq.astype(jnp.float32)
            s = jnp.where(eq, NEG_BIG, s)
            nbr = jnp.dot(pT, oh, preferred_element_type=jnp.float32)
            nb0, nb1, nb2 = nbr[0:1, :], nbr[1:2, :], nbr[2:3, :]
            rows = [
                nb0 - ctr0, ctr0, nb1 * ctr2 - nb2 * ctr1,
                nb1 - ctr1, ctr1, nb2 * ctr0 - nb0 * ctr2,
                nb2 - ctr2, ctr2, nb0 * ctr1 - nb1 * ctr0,
            ]
            x = jnp.concatenate(rows, axis=0)
            g = jnp.concatenate([
                rows[0] * rows[0] + rows[3] * rows[3] + rows[6] * rows[6],
                rows[1] * rows[1] + rows[4] * rows[4] + rows[7] * rows[7],
                rows[2] * rows[2] + rows[5] * rows[5] + rows[8] * rows[8],
                rows[0] * rows[1] + rows[3] * rows[4] + rows[6] * rows[7],
                rows[0] * rows[2] + rows[3] * rows[5] + rows[6] * rows[8],
                rows[1] * rows[2] + rows[4] * rows[5] + rows[7] * rows[8],
            ], axis=0)
            dd = jnp.dot(qt_ref[...], g, preferred_element_type=jnp.float32)
            yd = jnp.dot(wd9t_ref[...], x, preferred_element_type=jnp.float32)
            d0, d1, d2 = yd[0:64], yd[64:128], yd[128:192]
            dotv = dd[0:64]
            dnsq = d0 * d0 + d1 * d1 + d2 * d2
            sc = jnp.where(dotv < 0.0,
                           dotv * pl.reciprocal(dnsq + EPS, approx=True), 0.0)
            acc = acc + jnp.concatenate(
                [sc * d0, sc * d1, sc * d2], axis=0)
            xpool = xpool + x

        yp = jnp.dot(wp9t_ref[...], xpool, preferred_element_type=jnp.float32)
        out = (yp - acc) * (1.0 / k)
        o0_ref[bi] = out[0:64].T
        o1_ref[bi] = out[64:128].T
        o2_ref[bi] = out[128:192].T


def _knn_convpos(p, wd9t, wp9t, qt, k):
    B, N, _ = p.shape
    bb = 2
    pt = jnp.transpose(p, (0, 2, 1))
    out_sh = jax.ShapeDtypeStruct((B, N, 64), jnp.float32)
    return pl.pallas_call(
        functools.partial(_knn_convpos_kernel, n=N, k=k, bb=bb),
        out_shape=(out_sh, out_sh, out_sh),
        grid_spec=pltpu.PrefetchScalarGridSpec(
            num_scalar_prefetch=0,
            grid=(B // bb,),
            in_specs=[
                pl.BlockSpec((bb, N, 3), lambda i: (i, 0, 0)),
                pl.BlockSpec((bb, 3, N), lambda i: (i, 0, 0)),
                pl.BlockSpec((192, 9), lambda i: (0, 0)),
                pl.BlockSpec((192, 9), lambda i: (0, 0)),
                pl.BlockSpec((128, 6), lambda i: (0, 0)),
            ],
            out_specs=(
                pl.BlockSpec((bb, N, 64), lambda i: (i, 0, 0)),
                pl.BlockSpec((bb, N, 64), lambda i: (i, 0, 0)),
                pl.BlockSpec((bb, N, 64), lambda i: (i, 0, 0)),
            ),
        ),
        compiler_params=pltpu.CompilerParams(dimension_semantics=("parallel",)),
    )(p, pt, wd9t, wp9t, qt)


def _vn_lrelu(p, d):
    dot = jnp.sum(p * d, axis=0, keepdims=True)
    dnsq = jnp.sum(d * d, axis=0, keepdims=True)
    neg = (dot < 0.0).astype(jnp.float32)
    corr = dot * pl.reciprocal(dnsq + EPS, approx=True)
    return p - neg * corr * d


def _conv_chain_kernel(x0_ref, x1_ref, x2_ref, w1_ref, w2n_ref, w2p_ref,
                       w3n_ref, w3p_ref, o_ref, *, h, n_pts):
    bb = x0_ref.shape[0]
    inv_n = 1.0 / n_pts
    x = jnp.stack([x0_ref[...], x1_ref[...], x2_ref[...]], axis=0)
    x2d = x.reshape(3 * bb * n_pts, x.shape[-1])
    y = jnp.dot(x2d, w1_ref[...], preferred_element_type=jnp.float32)
    y = y.reshape(3, bb * n_pts, 2 * h)
    net = _vn_lrelu(y[..., :h], y[..., h:])

    for wn_ref, wp_ref in ((w2n_ref, w2p_ref), (w3n_ref, w3p_ref)):
        pooled = jnp.sum(net.reshape(3, bb, n_pts, h), axis=2) * inv_n
        pproj = jnp.dot(pooled.reshape(3 * bb, h), wp_ref[...],
                        preferred_element_type=jnp.float32)
        y = jnp.dot(net.reshape(3 * bb * n_pts, h), wn_ref[...],
                    preferred_element_type=jnp.float32)
        y = y.reshape(3, bb, n_pts, 2 * h) + pproj.reshape(3, bb, 1, 2 * h)
        y = y.reshape(3, bb * n_pts, 2 * h)
        net = _vn_lrelu(y[..., :h], y[..., h:])

    pooled = jnp.sum(net.reshape(3, bb, n_pts, h), axis=2, keepdims=True) * inv_n
    o_ref[...] = jnp.broadcast_to(pooled, o_ref.shape)


def _conv_chain(net0, net1, net2, w1, w2, w3):
    B, N, _ = net0.shape
    h = HID
    w2n, w2p = w2[:h], w2[h:]
    w3n, w3p = w3[:h], w3[h:]
    bb = 8

    out = pl.pallas_call(
        functools.partial(_conv_chain_kernel, h=h, n_pts=N),
        out_shape=jax.ShapeDtypeStruct((3, B, 8, h), jnp.float32),
        grid_spec=pltpu.PrefetchScalarGridSpec(
            num_scalar_prefetch=0,
            grid=(B // bb,),
            in_specs=[
                pl.BlockSpec((bb, N, 64), lambda i: (i, 0, 0)),
                pl.BlockSpec((bb, N, 64), lambda i: (i, 0, 0)),
                pl.BlockSpec((bb, N, 64), lambda i: (i, 0, 0)),
                pl.BlockSpec((64, 2 * h), lambda i: (0, 0)),
                pl.BlockSpec((h, 2 * h), lambda i: (0, 0)),
                pl.BlockSpec((h, 2 * h), lambda i: (0, 0)),
                pl.BlockSpec((h, 2 * h), lambda i: (0, 0)),
                pl.BlockSpec((h, 2 * h), lambda i: (0, 0)),
            ],
            out_specs=pl.BlockSpec((3, bb, 8, h), lambda i: (0, i, 0, 0)),
        ),
        compiler_params=pltpu.CompilerParams(dimension_semantics=("parallel",)),
    )(net0, net1, net2, w1, w2n, w2p, w3n, w3p)
    return out[:, :, 0, :]


def _vnt_normalize(w):
    return w / jnp.sum(w, axis=1, keepdims=True)


def _concat_w(wf, wd):
    return jnp.concatenate([_vnt_normalize(wf).T, _vnt_normalize(wd).T], axis=1)


def _build_convpos_mats(wf, wd):
    wfn = _vnt_normalize(wf).T
    wdn = _vnt_normalize(wd).T

    def blockdiag(w3):
        z = jnp.zeros((64, 3), jnp.float32)
        rows = []
        for v in range(3):
            row = [w3.T if v == vp else z for vp in range(3)]
            rows.append(jnp.concatenate(row, axis=1))
        return jnp.concatenate(rows, axis=0)

    pairs = [(0, 0), (1, 1), (2, 2), (0, 1), (0, 2), (1, 2)]
    qdot, qdn = [], []
    for c, cp in pairs:
        if c == cp:
            qdot.append(wfn[c] * wdn[c])
            qdn.append(wdn[c] * wdn[c])
        else:
            qdot.append(wfn[c] * wdn[cp] + wfn[cp] * wdn[c])
            qdn.append(2.0 * wdn[c] * wdn[cp])
    qt = jnp.concatenate([jnp.stack(qdot, axis=1),
                          jnp.stack(qdn, axis=1)], axis=0)
    return blockdiag(wdn), blockdiag(wfn), qt


def _forward(p, wd9t, wp9t, qt, w1, w2, w3):
    net0, net1, net2 = _knn_convpos(p, wd9t, wp9t, qt, K_NBRS)
    out = _conv_chain(net0, net1, net2, w1, w2, w3)
    return jnp.transpose(out, (1, 2, 0))


def kernel(p, conv_pos_wf, conv_pos_wd, conv1_wf, conv1_wd,
           conv2_wf, conv2_wd, conv3_wf, conv3_wd):
    wd9t, wp9t, qt = _build_convpos_mats(conv_pos_wf, conv_pos_wd)
    w1 = _concat_w(conv1_wf, conv1_wd)
    w2 = _concat_w(conv2_wf, conv2_wd)
    w3 = _concat_w(conv3_wf, conv3_wd)
    am = jax.sharding.get_abstract_mesh()
    if am is not None and not am.empty:
        mesh, axis = am, am.axis_names[0]
    else:
        mesh, axis = jax.sharding.Mesh(np.array(jax.devices()), ("b",)), "b"
    pspec = jax.sharding.PartitionSpec
    wspec = pspec(None, None)
    f = jax.shard_map(
        _forward, mesh=mesh,
        in_specs=(pspec(axis), wspec, wspec, wspec, wspec, wspec, wspec),
        out_specs=pspec(axis),
        check_vma=False,
    )
    return f(p, wd9t, wp9t, qt, w1, w2, w3)


# speedup vs baseline: 1.0379x; 1.0379x over previous
import functools

import jax
import jax.numpy as jnp
import numpy as np
from jax.experimental import pallas as pl
from jax.experimental.pallas import tpu as pltpu

EPS = 1e-6
K_NBRS = 20
HID = 128
NEG_BIG = -3.0e38


def _knn_convpos_kernel(p_ref, pt_ref, w24t_ref,
                        o0_ref, o1_ref, o2_ref, *, n, k, bb):
    for bi in range(bb):
        p2d = p_ref[bi]
        pt8 = pt_ref[bi]
        pT = pt8[0:3]

        inner = jnp.dot(p2d.astype(jnp.bfloat16), pT.astype(jnp.bfloat16),
                        preferred_element_type=jnp.float32)
        xx = pT * pT
        xxs = (xx[0:1, :] + xx[1:2, :]) + xx[2:3, :]
        xxc = jnp.sum(p2d * p2d, axis=1, keepdims=True)
        s = 2.0 * inner - xxc - xxs

        r1c = pltpu.roll(pt8, 7, 0)
        r2c = pltpu.roll(pt8, 6, 0)

        acc0 = jnp.zeros((64, n), jnp.float32)
        acc1 = jnp.zeros((64, n), jnp.float32)
        acc2 = jnp.zeros((64, n), jnp.float32)
        for _ in range(k):
            mx = jnp.max(s, axis=0, keepdims=True)
            eq = s == mx
            oh = eq.astype(jnp.float32)
            s = jnp.where(eq, NEG_BIG, s)
            nbr8 = jnp.dot(pt8, oh, preferred_element_type=jnp.float32)
            diff8 = nbr8 - pt8
            r1n = pltpu.roll(nbr8, 7, 0)
            r2n = pltpu.roll(nbr8, 6, 0)
            cross8 = r1n * r2c - r2n * r1c
            x24 = jnp.concatenate([diff8, pt8, cross8], axis=0)
            y = jnp.dot(w24t_ref[...], x24, preferred_element_type=jnp.float32)
            p0, d0 = y[0:64], y[64:128]
            p1, d1 = y[128:192], y[192:256]
            p2, d2 = y[256:320], y[320:384]
            dotv = p0 * d0 + p1 * d1 + p2 * d2
            dnsq = d0 * d0 + d1 * d1 + d2 * d2
            sc = jnp.where(dotv < 0.0,
                           dotv * pl.reciprocal(dnsq + EPS, approx=True), 0.0)
            acc0 = acc0 + (p0 - sc * d0)
            acc1 = acc1 + (p1 - sc * d1)
            acc2 = acc2 + (p2 - sc * d2)

        o0_ref[bi] = (acc0 * (1.0 / k)).T
        o1_ref[bi] = (acc1 * (1.0 / k)).T
        o2_ref[bi] = (acc2 * (1.0 / k)).T


def _knn_convpos(p, w24t, k):
    B, N, _ = p.shape
    bb = 2
    pt = jnp.transpose(p, (0, 2, 1))
    pt8 = jnp.tile(pt, (1, 3, 1))[:, :8]
    out_sh = jax.ShapeDtypeStruct((B, N, 64), jnp.float32)
    return pl.pallas_call(
        functools.partial(_knn_convpos_kernel, n=N, k=k, bb=bb),
        out_shape=(out_sh, out_sh, out_sh),
        grid_spec=pltpu.PrefetchScalarGridSpec(
            num_scalar_prefetch=0,
            grid=(B // bb,),
            in_specs=[
                pl.BlockSpec((bb, N, 3), lambda i: (i, 0, 0)),
                pl.BlockSpec((bb, 8, N), lambda i: (i, 0, 0)),
                pl.BlockSpec((384, 24), lambda i: (0, 0)),
            ],
            out_specs=(
                pl.BlockSpec((bb, N, 64), lambda i: (i, 0, 0)),
                pl.BlockSpec((bb, N, 64), lambda i: (i, 0, 0)),
                pl.BlockSpec((bb, N, 64), lambda i: (i, 0, 0)),
            ),
        ),
        compiler_params=pltpu.CompilerParams(dimension_semantics=("parallel",)),
    )(p, pt8, w24t)


def _vn_lrelu(p, d):
    dot = jnp.sum(p * d, axis=0, keepdims=True)
    dnsq = jnp.sum(d * d, axis=0, keepdims=True)
    neg = (dot < 0.0).astype(jnp.float32)
    corr = dot * pl.reciprocal(dnsq + EPS, approx=True)
    return p - neg * corr * d


def _conv_chain_kernel(x0_ref, x1_ref, x2_ref, w1_ref, w2n_ref, w2p_ref,
                       w3n_ref, w3p_ref, o_ref, *, h, n_pts):
    bb = x0_ref.shape[0]
    inv_n = 1.0 / n_pts
    x = jnp.stack([x0_ref[...], x1_ref[...], x2_ref[...]], axis=0)
    x2d = x.reshape(3 * bb * n_pts, x.shape[-1])
    y = jnp.dot(x2d, w1_ref[...], preferred_element_type=jnp.float32)
    y = y.reshape(3, bb * n_pts, 2 * h)
    net = _vn_lrelu(y[..., :h], y[..., h:])

    for wn_ref, wp_ref in ((w2n_ref, w2p_ref), (w3n_ref, w3p_ref)):
        pooled = jnp.sum(net.reshape(3, bb, n_pts, h), axis=2) * inv_n
        pproj = jnp.dot(pooled.reshape(3 * bb, h), wp_ref[...],
                        preferred_element_type=jnp.float32)
        y = jnp.dot(net.reshape(3 * bb * n_pts, h), wn_ref[...],
                    preferred_element_type=jnp.float32)
        y = y.reshape(3, bb, n_pts, 2 * h) + pproj.reshape(3, bb, 1, 2 * h)
        y = y.reshape(3, bb * n_pts, 2 * h)
        net = _vn_lrelu(y[..., :h], y[..., h:])

    pooled = jnp.sum(net.reshape(3, bb, n_pts, h), axis=2, keepdims=True) * inv_n
    o_ref[...] = jnp.broadcast_to(pooled, o_ref.shape)


def _conv_chain(net0, net1, net2, w1, w2, w3):
    B, N, _ = net0.shape
    h = HID
    w2n, w2p = w2[:h], w2[h:]
    w3n, w3p = w3[:h], w3[h:]
    bb = 8

    out = pl.pallas_call(
        functools.partial(_conv_chain_kernel, h=h, n_pts=N),
        out_shape=jax.ShapeDtypeStruct((3, B, 8, h), jnp.float32),
        grid_spec=pltpu.PrefetchScalarGridSpec(
            num_scalar_prefetch=0,
            grid=(B // bb,),
            in_specs=[
                pl.BlockSpec((bb, N, 64), lambda i: (i, 0, 0)),
                pl.BlockSpec((bb, N, 64), lambda i: (i, 0, 0)),
                pl.BlockSpec((bb, N, 64), lambda i: (i, 0, 0)),
                pl.BlockSpec((64, 2 * h), lambda i: (0, 0)),
                pl.BlockSpec((h, 2 * h), lambda i: (0, 0)),
                pl.BlockSpec((h, 2 * h), lambda i: (0, 0)),
                pl.BlockSpec((h, 2 * h), lambda i: (0, 0)),
                pl.BlockSpec((h, 2 * h), lambda i: (0, 0)),
            ],
            out_specs=pl.BlockSpec((3, bb, 8, h), lambda i: (0, i, 0, 0)),
        ),
        compiler_params=pltpu.CompilerParams(dimension_semantics=("parallel",)),
    )(net0, net1, net2, w1, w2n, w2p, w3n, w3p)
    return out[:, :, 0, :]


def _vnt_normalize(w):
    return w / jnp.sum(w, axis=1, keepdims=True)


def _concat_w(wf, wd):
    return jnp.concatenate([_vnt_normalize(wf).T, _vnt_normalize(wd).T], axis=1)


def _build_w24t(wf, wd):
    w = _concat_w(wf, wd)
    out = jnp.zeros((384, 24), jnp.float32)
    for v in range(3):
        for b in range(3):
            out = out.at[v * 128:(v + 1) * 128, 8 * b + v].set(w[b])
    return out


def _forward(p, w24t, w1, w2, w3):
    net0, net1, net2 = _knn_convpos(p, w24t, K_NBRS)
    out = _conv_chain(net0, net1, net2, w1, w2, w3)
    return jnp.transpose(out, (1, 2, 0))


def kernel(p, conv_pos_wf, conv_pos_wd, conv1_wf, conv1_wd,
           conv2_wf, conv2_wd, conv3_wf, conv3_wd):
    w24t = _build_w24t(conv_pos_wf, conv_pos_wd)
    w1 = _concat_w(conv1_wf, conv1_wd)
    w2 = _concat_w(conv2_wf, conv2_wd)
    w3 = _concat_w(conv3_wf, conv3_wd)
    am = jax.sharding.get_abstract_mesh()
    if am is not None and not am.empty:
        mesh, axis = am, am.axis_names[0]
    else:
        mesh, axis = jax.sharding.Mesh(np.array(jax.devices()), ("b",)), "b"
    pspec = jax.sharding.PartitionSpec
    wspec = pspec(None, None)
    f = jax.shard_map(
        _forward, mesh=mesh,
        in_specs=(pspec(axis), wspec, wspec, wspec, wspec),
        out_specs=pspec(axis),
        check_vma=False,
    )
    return f(p, w24t, w1, w2, w3)
